# Initial kernel scaffold; baseline (speedup 1.0000x reference)
import jax
import jax.numpy as jnp
from jax.experimental import pallas as pl


def kernel(atom_features, atom_partition_indices, recurrent_kernel, bias):
    raise NotImplementedError("write your pallas kernel here")


# speedup vs baseline: 4.3448x; 4.3448x over previous
import functools

import jax
import jax.numpy as jnp
from jax import lax
from jax.experimental import pallas as pl
from jax.experimental.pallas import tpu as pltpu
from jax.experimental.pallas import tpu_sc as plsc

N = 100000
D = 128
B = 1024
STEPS = 8
NW = 32
SEG_PER_W = B // NW
ROW_OUT = D + 16


def _sc_attention_body(a_hbm, c_hbm, off_hbm, out_hbm,
                       a_buf, c_buf, off_buf, stage):
    cid = lax.axis_index("c")
    sid = lax.axis_index("s")
    w = cid * 16 + sid
    seg0 = w * SEG_PER_W

    pltpu.sync_copy(c_hbm.at[pl.ds(seg0, SEG_PER_W)], c_buf)
    pltpu.sync_copy(off_hbm.at[pl.ds(seg0, 48)], off_buf)

    iota = lax.iota(jnp.int32, 16)

    def off_at(i):
        v = plsc.load_gather(off_buf, [jnp.full((16,), 0, jnp.int32) + i])
        return v[0]

    def seg_body(s_local, _):
        start = off_at(s_local)
        end = off_at(s_local + 1)
        nchunks = lax.div(end - start + 15, jnp.int32(16))
        c_vecs = [c_buf[s_local, pl.ds(k * 16, 16)] for k in range(D // 16)]

        def chunk_body(j, carry):
            den_acc = carry[0]
            num_acc = carry[1:]
            base = start + j * 16
            base_c = jnp.minimum(base, N - 16)
            pltpu.sync_copy(a_hbm.at[pl.ds(base_c, 16)], a_buf)
            ids = base_c + iota
            valid = (ids >= base) & (ids < end)
            r = jnp.zeros((16,), jnp.float32)
            for d in range(D):
                a_d = plsc.load_gather(
                    a_buf, [iota, jnp.full((16,), d, jnp.int32)])
                r = r + a_d * c_vecs[d // 16][d % 16]
            e = jnp.where(valid, jnp.exp(r), 0.0)
            den_acc = den_acc + jnp.sum(e)
            new_num = list(num_acc)
            for i in range(16):
                e_i = e[i]
                for k in range(D // 16):
                    new_num[k] = new_num[k] + a_buf[i, pl.ds(k * 16, 16)] * e_i
            return (den_acc, *new_num)

        init = (jnp.float32(0.0),) + tuple(
            jnp.zeros((16,), jnp.float32) for _ in range(D // 16))
        res = lax.fori_loop(0, nchunks, chunk_body, init)
        den = res[0]
        for k in range(D // 16):
            stage[s_local, pl.ds(k * 16, 16)] = res[1 + k]
        stage[s_local, pl.ds(D, 16)] = jnp.where(
            iota == 0, den, 0.0)
        return _

    lax.fori_loop(0, SEG_PER_W, seg_body, 0)
    pltpu.sync_copy(stage, out_hbm.at[pl.ds(seg0, SEG_PER_W)])


@jax.jit
def _sc_attention(atom_features, carry, offsets):
    mesh = plsc.VectorSubcoreMesh(core_axis_name="c", subcore_axis_name="s")
    f = pl.kernel(
        _sc_attention_body,
        out_type=jax.ShapeDtypeStruct((B, ROW_OUT), jnp.float32),
        mesh=mesh,
        scratch_types=[
            pltpu.VMEM((16, D), jnp.float32),
            pltpu.VMEM((SEG_PER_W, D), jnp.float32),
            pltpu.VMEM((48,), jnp.int32),
            pltpu.VMEM((SEG_PER_W, ROW_OUT), jnp.float32),
        ],
        compiler_params=pltpu.CompilerParams(
            use_tc_tiling_on_sc=False, needs_layout_passes=False),
    )
    return f(atom_features, carry, offsets)


def _tc_lstm_body(m_ref, c_ref, p_ref, w_ref, b_ref,
                  m_out, c_out, ce_out):
    num = p_ref[:, :D]
    den = p_ref[:, D:D + 1]
    readout = num / jnp.maximum(den, 1e-30)
    c = c_ref[...]
    z = (
        jnp.dot(c, w_ref[:D, :], preferred_element_type=jnp.float32)
        + jnp.dot(readout, w_ref[D:, :], preferred_element_type=jnp.float32)
        + b_ref[...]
    )
    u = jax.nn.sigmoid(z[:, :D])
    f = jax.nn.sigmoid(z[:, D:2 * D])
    g = jnp.tanh(z[:, 2 * D:3 * D])
    o = jax.nn.sigmoid(z[:, 3 * D:])
    m_new = f * m_ref[...] + u * g
    m_out[...] = m_new
    c_out[...] = o * jnp.tanh(m_new)
    ce_out[:, :D] = c
    ce_out[:, D:] = readout


@jax.jit
def _tc_lstm(m, c, parts, w, b2d):
    return pl.pallas_call(
        _tc_lstm_body,
        out_shape=(
            jax.ShapeDtypeStruct((B, D), jnp.float32),
            jax.ShapeDtypeStruct((B, D), jnp.float32),
            jax.ShapeDtypeStruct((B, 2 * D), jnp.float32),
        ),
    )(m, c, parts, w, b2d)


def kernel(atom_features, atom_partition_indices, recurrent_kernel, bias):
    seg = atom_partition_indices
    offsets = jnp.searchsorted(
        seg, jnp.arange(B + 1, dtype=jnp.int32), side="left"
    ).astype(jnp.int32)
    offsets = jnp.concatenate(
        [offsets, jnp.full((15,), N, jnp.int32)])
    b2d = bias.reshape(1, 4 * D)
    m = jnp.zeros((B, D), jnp.float32)
    c = jnp.zeros((B, D), jnp.float32)
    ce = None
    for _ in range(STEPS):
        parts = _sc_attention(atom_features, c, offsets)
        m, c, ce = _tc_lstm(m, c, parts, recurrent_kernel, b2d)
    return ce


# speedup vs baseline: 6.7747x; 1.5593x over previous
import functools

import jax
import jax.numpy as jnp
from jax import lax
from jax.experimental import pallas as pl
from jax.experimental.pallas import tpu as pltpu
from jax.experimental.pallas import tpu_sc as plsc

N = 100000
D = 128
B = 1024
STEPS = 8
NW = 32
SEG_PER_W = B // NW
ROW_OUT = D + 16


def _sc_attention_body(a_hbm, c_hbm, off_hbm, out_hbm,
                       a_buf, c_buf, off_buf, stage, sem):
    cid = lax.axis_index("c")
    sid = lax.axis_index("s")
    w = cid * 16 + sid
    seg0 = w * SEG_PER_W

    pltpu.sync_copy(c_hbm.at[pl.ds(seg0, SEG_PER_W)], c_buf)
    pltpu.sync_copy(off_hbm.at[pl.ds(seg0, 48)], off_buf)

    iota = lax.iota(jnp.int32, 16)

    def off_at(i):
        v = plsc.load_gather(off_buf, [jnp.full((16,), 0, jnp.int32) + i])
        return v[0]

    def issue(base, p):
        pltpu.make_async_copy(
            a_hbm.at[pl.ds(base, 16)], a_buf.at[p], sem.at[p]).start()

    def wait(p):
        pltpu.make_async_copy(
            a_hbm.at[pl.ds(0, 16)], a_buf.at[p], sem.at[p]).wait()

    issue(jnp.minimum(off_at(0), N - 16), 0)

    def seg_body(s_local, parity):
        start = off_at(s_local)
        end = off_at(s_local + 1)
        nchunks = lax.div(end - start + 15, jnp.int32(16))
        c_vecs = [c_buf[s_local, pl.ds(k * 16, 16)] for k in range(D // 16)]

        def chunk_body(j, carry):
            den_acc = carry[0]
            num_acc = carry[1:-1]
            p = carry[-1]
            base = start + j * 16
            base_c = jnp.minimum(base, N - 16)
            ids = base_c + iota
            valid = (ids >= base) & (ids < end)
            wait(p)
            nxt = jnp.where(j + 1 < nchunks, base + 16, end)
            issue(jnp.minimum(nxt, N - 16), 1 - p)
            r = jnp.zeros((16,), jnp.float32)
            for d in range(D):
                a_d = plsc.load_gather(
                    a_buf, [jnp.full((16,), 0, jnp.int32) + p, iota,
                            jnp.full((16,), d, jnp.int32)])
                r = r + a_d * c_vecs[d // 16][d % 16]
            e = jnp.where(valid, jnp.exp(r), 0.0)
            den_acc = den_acc + jnp.sum(e)
            new_num = list(num_acc)
            for i in range(16):
                e_i = e[i]
                for k in range(D // 16):
                    new_num[k] = (new_num[k]
                                  + a_buf[p, i, pl.ds(k * 16, 16)] * e_i)
            return (den_acc, *new_num, 1 - p)

        init = (jnp.float32(0.0),) + tuple(
            jnp.zeros((16,), jnp.float32) for _ in range(D // 16)) + (parity,)
        res = lax.fori_loop(0, nchunks, chunk_body, init)
        den = res[0]
        for k in range(D // 16):
            stage[s_local, pl.ds(k * 16, 16)] = res[1 + k]
        stage[s_local, pl.ds(D, 16)] = jnp.where(
            iota == 0, den, 0.0)
        return res[-1]

    parity = lax.fori_loop(0, SEG_PER_W, seg_body, jnp.int32(0))
    wait(parity)
    pltpu.sync_copy(stage, out_hbm.at[pl.ds(seg0, SEG_PER_W)])


@jax.jit
def _sc_attention(atom_features, carry, offsets):
    mesh = plsc.VectorSubcoreMesh(core_axis_name="c", subcore_axis_name="s")
    f = pl.kernel(
        _sc_attention_body,
        out_type=jax.ShapeDtypeStruct((B, ROW_OUT), jnp.float32),
        mesh=mesh,
        scratch_types=[
            pltpu.VMEM((2, 16, D), jnp.float32),
            pltpu.VMEM((SEG_PER_W, D), jnp.float32),
            pltpu.VMEM((48,), jnp.int32),
            pltpu.VMEM((SEG_PER_W, ROW_OUT), jnp.float32),
            pltpu.SemaphoreType.DMA((2,)),
        ],
        compiler_params=pltpu.CompilerParams(
            use_tc_tiling_on_sc=False, needs_layout_passes=False),
    )
    return f(atom_features, carry, offsets)


def _tc_lstm_body(m_ref, c_ref, p_ref, w_ref, b_ref,
                  m_out, c_out, ce_out):
    num = p_ref[:, :D]
    den = p_ref[:, D:D + 1]
    readout = num / jnp.maximum(den, 1e-30)
    c = c_ref[...]
    z = (
        jnp.dot(c, w_ref[:D, :], preferred_element_type=jnp.float32)
        + jnp.dot(readout, w_ref[D:, :], preferred_element_type=jnp.float32)
        + b_ref[...]
    )
    u = jax.nn.sigmoid(z[:, :D])
    f = jax.nn.sigmoid(z[:, D:2 * D])
    g = jnp.tanh(z[:, 2 * D:3 * D])
    o = jax.nn.sigmoid(z[:, 3 * D:])
    m_new = f * m_ref[...] + u * g
    m_out[...] = m_new
    c_out[...] = o * jnp.tanh(m_new)
    ce_out[:, :D] = c
    ce_out[:, D:] = readout


@jax.jit
def _tc_lstm(m, c, parts, w, b2d):
    return pl.pallas_call(
        _tc_lstm_body,
        out_shape=(
            jax.ShapeDtypeStruct((B, D), jnp.float32),
            jax.ShapeDtypeStruct((B, D), jnp.float32),
            jax.ShapeDtypeStruct((B, 2 * D), jnp.float32),
        ),
    )(m, c, parts, w, b2d)


def kernel(atom_features, atom_partition_indices, recurrent_kernel, bias):
    seg = atom_partition_indices
    offsets = jnp.searchsorted(
        seg, jnp.arange(B + 1, dtype=jnp.int32), side="left"
    ).astype(jnp.int32)
    offsets = jnp.concatenate(
        [offsets, jnp.full((15,), N, jnp.int32)])
    b2d = bias.reshape(1, 4 * D)
    m = jnp.zeros((B, D), jnp.float32)
    c = jnp.zeros((B, D), jnp.float32)
    ce = None
    for _ in range(STEPS):
        parts = _sc_attention(atom_features, c, offsets)
        m, c, ce = _tc_lstm(m, c, parts, recurrent_kernel, b2d)
    return ce


# speedup vs baseline: 17.9070x; 2.6432x over previous
import functools

import jax
import jax.numpy as jnp
from jax import lax
from jax.experimental import pallas as pl
from jax.experimental.pallas import tpu as pltpu
from jax.experimental.pallas import tpu_sc as plsc

N = 100000
D = 128
B = 1024
STEPS = 8
NW = 32
SEG_PER_W = B // NW
ROW_OUT = D + 16
CHUNK = 64


def _sc_attention_body(a_hbm, c_hbm, off_hbm, out_hbm,
                       a_buf, c_buf, off_buf, stage, sem):
    cid = lax.axis_index("c")
    sid = lax.axis_index("s")
    w = cid * 16 + sid
    seg0 = w * SEG_PER_W

    pltpu.sync_copy(c_hbm.at[pl.ds(seg0, SEG_PER_W)], c_buf)
    pltpu.sync_copy(off_hbm.at[pl.ds(seg0, 48)], off_buf)

    iota = lax.iota(jnp.int32, 16)

    def off_at(i):
        v = plsc.load_gather(off_buf, [jnp.full((16,), 0, jnp.int32) + i])
        return v[0]

    def issue(base, p):
        pltpu.make_async_copy(
            a_hbm.at[pl.ds(base, CHUNK)], a_buf.at[p], sem.at[p]).start()

    def wait(p):
        pltpu.make_async_copy(
            a_hbm.at[pl.ds(0, CHUNK)], a_buf.at[p], sem.at[p]).wait()

    issue(jnp.minimum(off_at(0), N - CHUNK), 0)

    def seg_body(s_local, parity):
        start = off_at(s_local)
        end = off_at(s_local + 1)
        nchunks = lax.div(end - start + (CHUNK - 1), jnp.int32(CHUNK))
        c_vecs = [c_buf[s_local, pl.ds(k * 16, 16)] for k in range(D // 16)]

        def chunk_body(j, carry):
            den_v = carry[0]
            num = list(carry[1:-1])
            p = carry[-1]
            base = start + j * CHUNK
            base_c = jnp.minimum(base, N - CHUNK)
            wait(p)
            nxt = jnp.where(j + 1 < nchunks, base + CHUNK, end)
            issue(jnp.minimum(nxt, N - CHUNK), 1 - p)
            for i in range(CHUNK):
                row = [a_buf[p, i, pl.ds(k * 16, 16)] for k in range(D // 16)]
                prod = [row[k] * c_vecs[k] for k in range(D // 16)]
                t01 = (prod[0] + prod[1]) + (prod[2] + prod[3])
                t23 = (prod[4] + prod[5]) + (prod[6] + prod[7])
                r_s = jnp.sum(t01 + t23)
                aid = base_c + i
                valid = (aid >= base) & (aid < end)
                e_sp = jnp.where(
                    valid, jnp.exp(jnp.zeros((16,), jnp.float32) + r_s), 0.0)
                den_v = den_v + e_sp
                for k in range(D // 16):
                    num[k] = num[k] + row[k] * e_sp
            return (den_v, *num, 1 - p)

        init = tuple(
            jnp.zeros((16,), jnp.float32) for _ in range(D // 16 + 1)
        ) + (parity,)
        res = lax.fori_loop(0, nchunks, chunk_body, init)
        for k in range(D // 16):
            stage[s_local, pl.ds(k * 16, 16)] = res[1 + k]
        stage[s_local, pl.ds(D, 16)] = jnp.where(
            iota == 0, res[0], 0.0)
        return res[-1]

    parity = lax.fori_loop(0, SEG_PER_W, seg_body, jnp.int32(0))
    wait(parity)
    pltpu.sync_copy(stage, out_hbm.at[pl.ds(seg0, SEG_PER_W)])


@jax.jit
def _sc_attention(atom_features, carry, offsets):
    mesh = plsc.VectorSubcoreMesh(core_axis_name="c", subcore_axis_name="s")
    f = pl.kernel(
        _sc_attention_body,
        out_type=jax.ShapeDtypeStruct((B, ROW_OUT), jnp.float32),
        mesh=mesh,
        scratch_types=[
            pltpu.VMEM((2, CHUNK, D), jnp.float32),
            pltpu.VMEM((SEG_PER_W, D), jnp.float32),
            pltpu.VMEM((48,), jnp.int32),
            pltpu.VMEM((SEG_PER_W, ROW_OUT), jnp.float32),
            pltpu.SemaphoreType.DMA((2,)),
        ],
        compiler_params=pltpu.CompilerParams(
            use_tc_tiling_on_sc=False, needs_layout_passes=False),
    )
    return f(atom_features, carry, offsets)


def _tc_lstm_body(m_ref, c_ref, p_ref, w_ref, b_ref,
                  m_out, c_out, ce_out):
    num = p_ref[:, :D]
    den = p_ref[:, D:D + 1]
    readout = num / jnp.maximum(den, 1e-30)
    c = c_ref[...]
    z = (
        jnp.dot(c, w_ref[:D, :], preferred_element_type=jnp.float32)
        + jnp.dot(readout, w_ref[D:, :], preferred_element_type=jnp.float32)
        + b_ref[...]
    )
    u = jax.nn.sigmoid(z[:, :D])
    f = jax.nn.sigmoid(z[:, D:2 * D])
    g = jnp.tanh(z[:, 2 * D:3 * D])
    o = jax.nn.sigmoid(z[:, 3 * D:])
    m_new = f * m_ref[...] + u * g
    m_out[...] = m_new
    c_out[...] = o * jnp.tanh(m_new)
    ce_out[:, :D] = c
    ce_out[:, D:] = readout


@jax.jit
def _tc_lstm(m, c, parts, w, b2d):
    return pl.pallas_call(
        _tc_lstm_body,
        out_shape=(
            jax.ShapeDtypeStruct((B, D), jnp.float32),
            jax.ShapeDtypeStruct((B, D), jnp.float32),
            jax.ShapeDtypeStruct((B, 2 * D), jnp.float32),
        ),
    )(m, c, parts, w, b2d)


def kernel(atom_features, atom_partition_indices, recurrent_kernel, bias):
    seg = atom_partition_indices
    offsets = jnp.searchsorted(
        seg, jnp.arange(B + 1, dtype=jnp.int32), side="left"
    ).astype(jnp.int32)
    offsets = jnp.concatenate(
        [offsets, jnp.full((15,), N, jnp.int32)])
    b2d = bias.reshape(1, 4 * D)
    m = jnp.zeros((B, D), jnp.float32)
    c = jnp.zeros((B, D), jnp.float32)
    ce = None
    for _ in range(STEPS):
        parts = _sc_attention(atom_features, c, offsets)
        m, c, ce = _tc_lstm(m, c, parts, recurrent_kernel, b2d)
    return ce
